# baseline (speedup 1.0000x reference)
import functools

import jax
import jax.numpy as jnp
from jax import lax
from jax.experimental import pallas as pl
from jax.experimental.pallas import tpu as pltpu
from jax.experimental.pallas import tpu_sc as plsc

D_MIN = 0.001
BIG = 3e38

B = 4
NX = 256
IMG = 224
NSUB = 16
WPB = 8
ROWS = IMG // WPB
VPR = IMG // 16
BINS = 272
CPAD = 528
NCELL = 1024
RLO = NCELL + 32


def _chamfer_body(cu_hbm, tp_hbm, out_hbm, cu_all, cu_v, cs_v, cpad_v, rlo_v,
                  y_v, hi_v, lo_v, buf_v, below_v, st_v, sml_v, out_v,
                  sem, val_sh, rnk_sh, hi_sh, lo_sh, st_sh):
    c = lax.axis_index("c")
    s = lax.axis_index("s")
    batch = c * 2 + s // WPB
    chunk = s % WPB

    g3 = 24 + jnp.bitwise_and(chunk, 3)
    dmas = [
        pltpu.async_copy(
            tp_hbm.at[batch, pl.ds(g * 8, 8), :],
            y_v.at[pl.ds(i * 8, 8), :], sem)
        for i, g in enumerate((chunk, chunk + 8, chunk + 16, g3))
    ]
    pltpu.sync_copy(cu_hbm, cu_all)

    lane = lax.iota(jnp.int32, 16)
    zeros_i = jnp.zeros((16,), jnp.int32)
    zeros_f = jnp.zeros((16,), jnp.float32)
    ones_i = jnp.full((16,), 1, jnp.int32)
    neg_big = jnp.full((16,), -BIG, jnp.float32)
    pos_big = jnp.full((16,), BIG, jnp.float32)

    for ch in range(NX // 16):
        cu_v[pl.ds(ch * 16, 16)] = cu_all[batch, pl.ds(ch * 16, 16)]

    g0_idx = chunk * 32 + lane
    g1_idx = g0_idx + 16
    a0 = plsc.load_gather(cu_v, [g0_idx])
    a1 = plsc.load_gather(cu_v, [g1_idx])

    def rank_step(t, carry):
        r0, r1 = carry
        for d in range(0, NX, NX // 4):
            i0 = jnp.bitwise_and(g0_idx + (t + d), NX - 1)
            i1 = jnp.bitwise_and(g1_idx + (t + d), NX - 1)
            w0 = plsc.load_gather(cu_v, [i0])
            w1 = plsc.load_gather(cu_v, [i1])
            c0 = jnp.logical_or(w0 < a0,
                                jnp.logical_and(w0 == a0, i0 < g0_idx))
            c1 = jnp.logical_or(w1 < a1,
                                jnp.logical_and(w1 == a1, i1 < g1_idx))
            r0 = r0 + jnp.where(c0, ones_i, zeros_i)
            r1 = r1 + jnp.where(c1, ones_i, zeros_i)
        return r0, r1

    r0, r1 = lax.fori_loop(1, NX // 4 + 1, rank_step, (zeros_i, zeros_i))

    sml_v[pl.ds(0, 16)] = a0
    sml_v[pl.ds(16, 16)] = a1
    pltpu.sync_copy(sml_v.at[pl.ds(0, 32)], val_sh.at[pl.ds(s * 32, 32)])
    sml_v[pl.ds(0, 16)] = plsc.bitcast(r0, jnp.float32)
    sml_v[pl.ds(16, 16)] = plsc.bitcast(r1, jnp.float32)
    pltpu.sync_copy(sml_v.at[pl.ds(0, 32)], rnk_sh.at[pl.ds(s * 32, 32)])
    plsc.subcore_barrier()

    grp = (s // WPB) * WPB * 32
    pltpu.sync_copy(val_sh.at[pl.ds(grp, NX)], buf_v.at[pl.ds(0, NX)])
    pltpu.sync_copy(rnk_sh.at[pl.ds(grp, NX)], buf_v.at[pl.ds(NX, NX)])
    for ch in range(NX // 16):
        vals = buf_v[pl.ds(ch * 16, 16)]
        rnks = plsc.bitcast(buf_v[pl.ds(NX + ch * 16, 16)], jnp.int32)
        plsc.store_scatter(cs_v, [rnks], vals)

    def build_cpad(i, _):
        gidx = i * 16 + lane
        src = jnp.minimum(jnp.maximum(gidx - 1, 0), NX - 1)
        g = plsc.load_gather(cs_v, [src])
        v = jnp.where(gidx == 0, neg_big, jnp.where(gidx >= NX + 1, pos_big, g))
        cpad_v[pl.ds(pl.multiple_of(i * 16, 16), 16)] = v
        return 0

    lax.fori_loop(0, CPAD // 16, build_cpad, 0)

    def zero_rlo(i, _):
        rlo_v[pl.ds(pl.multiple_of(i * 16, 16), 16)] = zeros_i
        return 0

    lax.fori_loop(0, RLO // 16, zero_rlo, 0)
    for ch in range(NX // 16):
        cc = cs_v[pl.ds(ch * 16, 16)]
        q = jnp.minimum(jnp.maximum(
            (cc * float(NCELL)).astype(jnp.int32), 0), NCELL - 1)
        plsc.addupdate_scatter(rlo_v, [q], ones_i)
    last_idx = jnp.full((16,), 15, jnp.int32)

    def psum_step(i, st):
        carry, m = st
        h = rlo_v[pl.ds(pl.multiple_of(i * 16, 16), 16)]
        incl = plsc.cumsum(h)
        rlo_v[pl.ds(pl.multiple_of(i * 16, 16), 16)] = incl - h + carry
        last = incl.at[last_idx].get(mode="promise_in_bounds")
        return carry + last, jnp.maximum(m, h)

    _, maxh = lax.fori_loop(0, RLO // 16, psum_step, (zeros_i, zeros_i))
    maxh_s = lax.reduce_max(maxh, axes=(0,))

    k_start = lax.while_loop(
        lambda k: k < maxh_s + 1, lambda k: k * 2, jnp.int32(1)) // 2
    kvecs = [jnp.full((16,), kc, jnp.int32) for kc in (4, 2, 1)]

    def init_bins(i, _):
        for u in range(16):
            off = pl.multiple_of(i * 256 + u * 16, 16)
            hi_v[pl.ds(off, 16)] = neg_big
            lo_v[pl.ds(off, 16)] = pos_big
        return 0

    lax.fori_loop(0, NSUB * BINS // 256, init_bins, 0)

    row_base = lane * BINS
    for d in dmas:
        d.wait()

    HALF = VPR // 2

    def point_pass(j, carry):
        acc, cnt = carry
        ridx = j >> 1
        row = ridx + jnp.where(jnp.logical_and(ridx >= 24, chunk >= 4), 4, 0)
        off = jnp.bitwise_and(j, 1) * (HALF * 16)
        ys = [y_v[row, pl.ds(off + t * 16, 16)] for t in range(HALF)]
        cells = [(y * float(NCELL)).astype(jnp.int32) for y in ys]
        rs = [plsc.load_gather(rlo_v, [cell]) for cell in cells]

        def search_head(state):
            k, r_list = state
            kv = jnp.broadcast_to(k, (16,))
            new_r = []
            for t in range(HALF):
                probe = plsc.load_gather(cpad_v, [r_list[t] + kv])
                new_r.append(
                    r_list[t] + jnp.where(probe <= ys[t], kv, zeros_i))
            return k // 2, tuple(new_r)

        k_head, rs = lax.while_loop(lambda st: st[0] >= 8, search_head,
                                    (k_start, tuple(rs)))
        rs = list(rs)

        for kv in kvecs:
            for t in range(HALF):
                probe = plsc.load_gather(cpad_v, [rs[t] + kv])
                take = probe <= ys[t]
                rs[t] = rs[t] + jnp.where(take, kv, zeros_i)

        for t in range(HALF):
            y, r = ys[t], rs[t]
            below = plsc.load_gather(cpad_v, [r])
            above = plsc.load_gather(cpad_v, [r + 1])
            valid = y >= D_MIN
            d = jnp.minimum(y - below, above - y)
            acc = acc + jnp.where(valid, d * d, zeros_f)
            cnt = cnt + jnp.where(valid, 1.0, zeros_f)
            hi_idx = row_base + r
            cur_hi = plsc.load_gather(hi_v, [hi_idx])
            plsc.store_scatter(hi_v, [hi_idx], jnp.maximum(cur_hi, y),
                               mask=valid)
            cur_lo = plsc.load_gather(lo_v, [hi_idx])
            plsc.store_scatter(lo_v, [hi_idx], jnp.minimum(cur_lo, y),
                               mask=valid)
        return acc, cnt

    acc, cnt = lax.fori_loop(0, 2 * ROWS, point_pass, (zeros_f, zeros_f))

    def fold_rows(ch, _):
        base = pl.multiple_of(ch * 16, 16)
        h = hi_v[pl.ds(base, 16)]
        l = lo_v[pl.ds(base, 16)]
        for w in range(1, NSUB):
            off = pl.multiple_of(w * BINS + ch * 16, 16)
            h = jnp.maximum(h, hi_v[pl.ds(off, 16)])
            l = jnp.minimum(l, lo_v[pl.ds(off, 16)])
        hi_v[pl.ds(base, 16)] = h
        lo_v[pl.ds(base, 16)] = l
        return 0

    lax.fori_loop(0, BINS // 16, fold_rows, 0)

    st_v[pl.ds(0, 16)] = acc
    st_v[pl.ds(16, 16)] = cnt

    pltpu.sync_copy(hi_v.at[pl.ds(0, BINS)], hi_sh.at[pl.ds(s * BINS, BINS)])
    pltpu.sync_copy(lo_v.at[pl.ds(0, BINS)], lo_sh.at[pl.ds(s * BINS, BINS)])
    pltpu.sync_copy(st_v, st_sh.at[pl.ds(s * 32, 32)])
    plsc.subcore_barrier()

    @pl.when(jnp.logical_or(s == 0, s == WPB))
    def _finalize():
        g0 = s
        pltpu.sync_copy(hi_sh.at[pl.ds(g0 * BINS, WPB * BINS)], buf_v)

        def fold_hi(ch, _):
            base = pl.multiple_of(ch * 16, 16)
            h = buf_v[pl.ds(base, 16)]
            for w in range(1, WPB):
                h = jnp.maximum(
                    h, buf_v[pl.ds(pl.multiple_of(w * BINS, 16) + base, 16)])
            hi_v[pl.ds(base, 16)] = h
            return 0

        lax.fori_loop(0, BINS // 16, fold_hi, 0)
        pltpu.sync_copy(lo_sh.at[pl.ds(g0 * BINS, WPB * BINS)], buf_v)

        def fold_lo(ch, _):
            base = pl.multiple_of(ch * 16, 16)
            l = buf_v[pl.ds(base, 16)]
            for w in range(1, WPB):
                l = jnp.minimum(
                    l, buf_v[pl.ds(pl.multiple_of(w * BINS, 16) + base, 16)])
            lo_v[pl.ds(base, 16)] = l
            return 0

        lax.fori_loop(0, BINS // 16, fold_lo, 0)
        pltpu.sync_copy(st_sh.at[pl.ds(g0 * 32, WPB * 32)],
                        buf_v.at[pl.ds(0, WPB * 32)])
        sv_acc = buf_v[pl.ds(0, 16)]
        sv_cnt = buf_v[pl.ds(16, 16)]
        for w in range(1, WPB):
            sv_acc = sv_acc + buf_v[pl.ds(w * 32, 16)]
            sv_cnt = sv_cnt + buf_v[pl.ds(w * 32 + 16, 16)]

        def prefix_step(ch, carry):
            v = hi_v[pl.ds(pl.multiple_of(ch * 16, 16), 16)]
            pm = jnp.maximum(plsc.cummax(v), carry)
            below_v[pl.ds(pl.multiple_of(ch * 16, 16), 16)] = pm
            return jnp.broadcast_to(lax.reduce_max(pm, axes=(0,)), (16,))

        lax.fori_loop(0, NX // 16, prefix_step, neg_big)

        def suffix_step(i, carry):
            accx, sufc = carry
            ch = NX // 16 - 1 - i
            base = pl.multiple_of(ch * 16, 16)
            v = lo_v[pl.ds(base + 1, 16)]
            rcm = plsc.cummax(-lax.rev(v, (0,)))
            suf = jnp.minimum(-lax.rev(rcm, (0,)), sufc)
            cvec = cs_v[pl.ds(base, 16)]
            bvec = below_v[pl.ds(base, 16)]
            dx = jnp.minimum(cvec - bvec, suf - cvec)
            accx = accx + dx * dx
            sufc = jnp.broadcast_to(lax.reduce_min(suf, axes=(0,)), (16,))
            return accx, sufc

        accx, _ = lax.fori_loop(0, NX // 16, suffix_step, (zeros_f, pos_big))

        chamx = jnp.broadcast_to(lax.reduce_sum(accx, axes=(0,)), (16,))
        chamy = jnp.broadcast_to(lax.reduce_sum(sv_acc, axes=(0,)), (16,))
        ycnt = jnp.broadcast_to(lax.reduce_sum(sv_cnt, axes=(0,)), (16,))
        val = (chamx * (1.0 / NX) + chamy / ycnt) * (1.0 / B)
        out_v[...] = jnp.where(lane == 0, val, zeros_f)
        pltpu.sync_copy(out_v, out_hbm.at[pl.ds(batch * 16, 16)])


@functools.partial(
    pl.kernel,
    out_type=jax.ShapeDtypeStruct((B * 16,), jnp.float32),
    mesh=plsc.VectorSubcoreMesh(core_axis_name="c", subcore_axis_name="s"),
    compiler_params=pltpu.CompilerParams(needs_layout_passes=False),
    scratch_types=dict(
        cu_all=pltpu.VMEM((B, NX), jnp.float32),
        cu_v=pltpu.VMEM((NX,), jnp.float32),
        cs_v=pltpu.VMEM((NX,), jnp.float32),
        cpad_v=pltpu.VMEM((CPAD,), jnp.float32),
        rlo_v=pltpu.VMEM((RLO,), jnp.int32),
        y_v=pltpu.VMEM((32, IMG), jnp.float32),
        hi_v=pltpu.VMEM((NSUB * BINS,), jnp.float32),
        lo_v=pltpu.VMEM((NSUB * BINS,), jnp.float32),
        buf_v=pltpu.VMEM((WPB * BINS,), jnp.float32),
        below_v=pltpu.VMEM((NX,), jnp.float32),
        st_v=pltpu.VMEM((32,), jnp.float32),
        sml_v=pltpu.VMEM((32,), jnp.float32),
        out_v=pltpu.VMEM((16,), jnp.float32),
        sem=pltpu.SemaphoreType.DMA,
        val_sh=pltpu.VMEM_SHARED((NSUB * 32,), jnp.float32),
        rnk_sh=pltpu.VMEM_SHARED((NSUB * 32,), jnp.float32),
        hi_sh=pltpu.VMEM_SHARED((NSUB * BINS,), jnp.float32),
        lo_sh=pltpu.VMEM_SHARED((NSUB * BINS,), jnp.float32),
        st_sh=pltpu.VMEM_SHARED((NSUB * 32,), jnp.float32),
    ),
)
def _chamfer_sc(cu_hbm, tp_hbm, out_hbm, *, cu_all, cu_v, cs_v, cpad_v, rlo_v,
                y_v, hi_v, lo_v, buf_v, below_v, st_v, sml_v, out_v,
                sem, val_sh, rnk_sh, hi_sh, lo_sh, st_sh):
    _chamfer_body(cu_hbm, tp_hbm, out_hbm, cu_all, cu_v, cs_v, cpad_v, rlo_v,
                  y_v, hi_v, lo_v, buf_v, below_v, st_v, sml_v, out_v,
                  sem, val_sh, rnk_sh, hi_sh, lo_sh, st_sh)


def kernel(bin_centers, target):
    if bin_centers.ndim == 1:
        bin_centers = bin_centers[None, :]
    cu = bin_centers.astype(jnp.float32)
    tp = target.astype(jnp.float32)
    out = _chamfer_sc(cu, tp)
    return jnp.sum(out)


# speedup vs baseline: 1.0221x; 1.0221x over previous
import functools

import jax
import jax.numpy as jnp
from jax import lax
from jax.experimental import pallas as pl
from jax.experimental.pallas import tpu as pltpu
from jax.experimental.pallas import tpu_sc as plsc

D_MIN = 0.001
BIG = 3e38

B = 4
NX = 256
IMG = 224
NSUB = 16
WPB = 8
ROWS = IMG // WPB
VPR = IMG // 16
BINS = 272
CPAD = 528
NCELL = 1024
RLO = NCELL + 32


def _chamfer_body(cu_hbm, tp_hbm, out_hbm, cu_all, cu_v, cs_v, cpad_v, rlo_v,
                  y_v, hi_v, lo_v, buf_v, below_v, st_v, sml_v, out_v,
                  sem, val_sh, rnk_sh, hi_sh, lo_sh, st_sh):
    c = lax.axis_index("c")
    s = lax.axis_index("s")
    batch = c * 2 + s // WPB
    chunk = s % WPB

    g3 = 24 + jnp.bitwise_and(chunk, 3)
    dmas = [
        pltpu.async_copy(
            tp_hbm.at[batch, pl.ds(g * 8, 8), :],
            y_v.at[pl.ds(i * 8, 8), :], sem)
        for i, g in enumerate((chunk, chunk + 8, chunk + 16, g3))
    ]
    pltpu.sync_copy(cu_hbm, cu_all)

    lane = lax.iota(jnp.int32, 16)
    zeros_i = jnp.zeros((16,), jnp.int32)
    zeros_f = jnp.zeros((16,), jnp.float32)
    ones_i = jnp.full((16,), 1, jnp.int32)
    neg_big = jnp.full((16,), -BIG, jnp.float32)
    pos_big = jnp.full((16,), BIG, jnp.float32)

    for ch in range(NX // 16):
        cu_v[pl.ds(ch * 16, 16)] = cu_all[batch, pl.ds(ch * 16, 16)]

    g0_idx = chunk * 32 + lane
    g1_idx = g0_idx + 16
    a0 = plsc.load_gather(cu_v, [g0_idx])
    a1 = plsc.load_gather(cu_v, [g1_idx])

    def rank_step(t, carry):
        r0, r1 = carry
        for d in range(0, NX, NX // 4):
            i0 = jnp.bitwise_and(g0_idx + (t + d), NX - 1)
            i1 = jnp.bitwise_and(g1_idx + (t + d), NX - 1)
            w0 = plsc.load_gather(cu_v, [i0])
            w1 = plsc.load_gather(cu_v, [i1])
            c0 = jnp.logical_or(w0 < a0,
                                jnp.logical_and(w0 == a0, i0 < g0_idx))
            c1 = jnp.logical_or(w1 < a1,
                                jnp.logical_and(w1 == a1, i1 < g1_idx))
            r0 = r0 + jnp.where(c0, ones_i, zeros_i)
            r1 = r1 + jnp.where(c1, ones_i, zeros_i)
        return r0, r1

    r0, r1 = lax.fori_loop(1, NX // 4 + 1, rank_step, (zeros_i, zeros_i))

    sml_v[pl.ds(0, 16)] = a0
    sml_v[pl.ds(16, 16)] = a1
    pltpu.sync_copy(sml_v.at[pl.ds(0, 32)], val_sh.at[pl.ds(s * 32, 32)])
    sml_v[pl.ds(0, 16)] = plsc.bitcast(r0, jnp.float32)
    sml_v[pl.ds(16, 16)] = plsc.bitcast(r1, jnp.float32)
    pltpu.sync_copy(sml_v.at[pl.ds(0, 32)], rnk_sh.at[pl.ds(s * 32, 32)])
    plsc.subcore_barrier()

    grp = (s // WPB) * WPB * 32
    pltpu.sync_copy(val_sh.at[pl.ds(grp, NX)], buf_v.at[pl.ds(0, NX)])
    pltpu.sync_copy(rnk_sh.at[pl.ds(grp, NX)], buf_v.at[pl.ds(NX, NX)])
    for ch in range(NX // 16):
        vals = buf_v[pl.ds(ch * 16, 16)]
        rnks = plsc.bitcast(buf_v[pl.ds(NX + ch * 16, 16)], jnp.int32)
        plsc.store_scatter(cs_v, [rnks], vals)

    def build_cpad(i, _):
        gidx = i * 16 + lane
        src = jnp.minimum(jnp.maximum(gidx - 1, 0), NX - 1)
        g = plsc.load_gather(cs_v, [src])
        v = jnp.where(gidx == 0, neg_big, jnp.where(gidx >= NX + 1, pos_big, g))
        cpad_v[pl.ds(pl.multiple_of(i * 16, 16), 16)] = v
        return 0

    lax.fori_loop(0, CPAD // 16, build_cpad, 0)

    def zero_rlo(i, _):
        rlo_v[pl.ds(pl.multiple_of(i * 16, 16), 16)] = zeros_i
        return 0

    lax.fori_loop(0, RLO // 16, zero_rlo, 0)
    for ch in range(NX // 16):
        cc = cs_v[pl.ds(ch * 16, 16)]
        q = jnp.minimum(jnp.maximum(
            (cc * float(NCELL)).astype(jnp.int32), 0), NCELL - 1)
        plsc.addupdate_scatter(rlo_v, [q], ones_i)
    last_idx = jnp.full((16,), 15, jnp.int32)

    def psum_step(i, st):
        carry, m = st
        h = rlo_v[pl.ds(pl.multiple_of(i * 16, 16), 16)]
        incl = plsc.cumsum(h)
        rlo_v[pl.ds(pl.multiple_of(i * 16, 16), 16)] = incl - h + carry
        last = incl.at[last_idx].get(mode="promise_in_bounds")
        return carry + last, jnp.maximum(m, h)

    _, maxh = lax.fori_loop(0, RLO // 16, psum_step, (zeros_i, zeros_i))
    maxh_s = lax.reduce_max(maxh, axes=(0,))

    k_start = lax.while_loop(
        lambda k: k < maxh_s + 1, lambda k: k * 2, jnp.int32(1)) // 2
    kvecs = [jnp.full((16,), kc, jnp.int32) for kc in (4, 2, 1)]

    def init_bins(i, _):
        for u in range(16):
            off = pl.multiple_of(i * 256 + u * 16, 16)
            hi_v[pl.ds(off, 16)] = neg_big
            lo_v[pl.ds(off, 16)] = pos_big
        return 0

    lax.fori_loop(0, NSUB * BINS // 256, init_bins, 0)

    row_base = lane * BINS
    for d in dmas:
        d.wait()

    def point_pass(j, carry):
        acc, cnt = carry
        row = j + jnp.where(jnp.logical_and(j >= 24, chunk >= 4), 4, 0)
        ys = [y_v[row, pl.ds(t * 16, 16)] for t in range(VPR)]
        cells = [(y * float(NCELL)).astype(jnp.int32) for y in ys]
        rs = [plsc.load_gather(rlo_v, [cell]) for cell in cells]

        def search_head(state):
            k, r_list = state
            kv = jnp.broadcast_to(k, (16,))
            new_r = []
            for t in range(VPR):
                probe = plsc.load_gather(cpad_v, [r_list[t] + kv])
                new_r.append(
                    r_list[t] + jnp.where(probe <= ys[t], kv, zeros_i))
            return k // 2, tuple(new_r)

        k_head, rs = lax.while_loop(lambda st: st[0] >= 8, search_head,
                                    (k_start, tuple(rs)))
        rs = list(rs)

        for kv in kvecs:
            for t in range(VPR):
                probe = plsc.load_gather(cpad_v, [rs[t] + kv])
                take = probe <= ys[t]
                rs[t] = rs[t] + jnp.where(take, kv, zeros_i)

        for t in range(VPR):
            y, r = ys[t], rs[t]
            below = plsc.load_gather(cpad_v, [r])
            above = plsc.load_gather(cpad_v, [r + 1])
            valid = y >= D_MIN
            d = jnp.minimum(y - below, above - y)
            acc = acc + jnp.where(valid, d * d, zeros_f)
            cnt = cnt + jnp.where(valid, 1.0, zeros_f)
            hi_idx = row_base + r
            cur_hi = plsc.load_gather(hi_v, [hi_idx])
            plsc.store_scatter(hi_v, [hi_idx], jnp.maximum(cur_hi, y),
                               mask=valid)
            cur_lo = plsc.load_gather(lo_v, [hi_idx])
            plsc.store_scatter(lo_v, [hi_idx], jnp.minimum(cur_lo, y),
                               mask=valid)
        return acc, cnt

    acc, cnt = lax.fori_loop(0, ROWS, point_pass, (zeros_f, zeros_f))

    def fold_rows(ch, _):
        base = pl.multiple_of(ch * 16, 16)
        h = hi_v[pl.ds(base, 16)]
        l = lo_v[pl.ds(base, 16)]
        for w in range(1, NSUB):
            off = pl.multiple_of(w * BINS + ch * 16, 16)
            h = jnp.maximum(h, hi_v[pl.ds(off, 16)])
            l = jnp.minimum(l, lo_v[pl.ds(off, 16)])
        hi_v[pl.ds(base, 16)] = h
        lo_v[pl.ds(base, 16)] = l
        return 0

    lax.fori_loop(0, BINS // 16, fold_rows, 0)

    st_v[pl.ds(0, 16)] = acc
    st_v[pl.ds(16, 16)] = cnt

    pltpu.sync_copy(hi_v.at[pl.ds(0, BINS)], hi_sh.at[pl.ds(s * BINS, BINS)])
    pltpu.sync_copy(lo_v.at[pl.ds(0, BINS)], lo_sh.at[pl.ds(s * BINS, BINS)])
    pltpu.sync_copy(st_v, st_sh.at[pl.ds(s * 32, 32)])
    plsc.subcore_barrier()

    @pl.when(jnp.logical_or(s == 0, s == WPB))
    def _finalize():
        g0 = s
        pltpu.sync_copy(hi_sh.at[pl.ds(g0 * BINS, WPB * BINS)], buf_v)

        def fold_hi(ch, _):
            base = pl.multiple_of(ch * 16, 16)
            h = buf_v[pl.ds(base, 16)]
            for w in range(1, WPB):
                h = jnp.maximum(
                    h, buf_v[pl.ds(pl.multiple_of(w * BINS, 16) + base, 16)])
            hi_v[pl.ds(base, 16)] = h
            return 0

        lax.fori_loop(0, BINS // 16, fold_hi, 0)
        pltpu.sync_copy(lo_sh.at[pl.ds(g0 * BINS, WPB * BINS)], buf_v)

        def fold_lo(ch, _):
            base = pl.multiple_of(ch * 16, 16)
            l = buf_v[pl.ds(base, 16)]
            for w in range(1, WPB):
                l = jnp.minimum(
                    l, buf_v[pl.ds(pl.multiple_of(w * BINS, 16) + base, 16)])
            lo_v[pl.ds(base, 16)] = l
            return 0

        lax.fori_loop(0, BINS // 16, fold_lo, 0)
        pltpu.sync_copy(st_sh.at[pl.ds(g0 * 32, WPB * 32)],
                        buf_v.at[pl.ds(0, WPB * 32)])
        sv_acc = buf_v[pl.ds(0, 16)]
        sv_cnt = buf_v[pl.ds(16, 16)]
        for w in range(1, WPB):
            sv_acc = sv_acc + buf_v[pl.ds(w * 32, 16)]
            sv_cnt = sv_cnt + buf_v[pl.ds(w * 32 + 16, 16)]

        def prefix_step(ch, carry):
            v = hi_v[pl.ds(pl.multiple_of(ch * 16, 16), 16)]
            pm = jnp.maximum(plsc.cummax(v), carry)
            below_v[pl.ds(pl.multiple_of(ch * 16, 16), 16)] = pm
            return jnp.broadcast_to(lax.reduce_max(pm, axes=(0,)), (16,))

        lax.fori_loop(0, NX // 16, prefix_step, neg_big)

        def suffix_step(i, carry):
            accx, sufc = carry
            ch = NX // 16 - 1 - i
            base = pl.multiple_of(ch * 16, 16)
            v = lo_v[pl.ds(base + 1, 16)]
            rcm = plsc.cummax(-lax.rev(v, (0,)))
            suf = jnp.minimum(-lax.rev(rcm, (0,)), sufc)
            cvec = cs_v[pl.ds(base, 16)]
            bvec = below_v[pl.ds(base, 16)]
            dx = jnp.minimum(cvec - bvec, suf - cvec)
            accx = accx + dx * dx
            sufc = jnp.broadcast_to(lax.reduce_min(suf, axes=(0,)), (16,))
            return accx, sufc

        accx, _ = lax.fori_loop(0, NX // 16, suffix_step, (zeros_f, pos_big))

        chamx = jnp.broadcast_to(lax.reduce_sum(accx, axes=(0,)), (16,))
        chamy = jnp.broadcast_to(lax.reduce_sum(sv_acc, axes=(0,)), (16,))
        ycnt = jnp.broadcast_to(lax.reduce_sum(sv_cnt, axes=(0,)), (16,))
        val = (chamx * (1.0 / NX) + chamy / ycnt) * (1.0 / B)
        out_v[...] = jnp.where(lane == 0, val, zeros_f)
        pltpu.sync_copy(out_v, out_hbm.at[pl.ds(batch * 16, 16)])


@functools.partial(
    pl.kernel,
    out_type=jax.ShapeDtypeStruct((B * 16,), jnp.float32),
    mesh=plsc.VectorSubcoreMesh(core_axis_name="c", subcore_axis_name="s"),
    compiler_params=pltpu.CompilerParams(needs_layout_passes=False),
    scratch_types=dict(
        cu_all=pltpu.VMEM((B, NX), jnp.float32),
        cu_v=pltpu.VMEM((NX,), jnp.float32),
        cs_v=pltpu.VMEM((NX,), jnp.float32),
        cpad_v=pltpu.VMEM((CPAD,), jnp.float32),
        rlo_v=pltpu.VMEM((RLO,), jnp.int32),
        y_v=pltpu.VMEM((32, IMG), jnp.float32),
        hi_v=pltpu.VMEM((NSUB * BINS,), jnp.float32),
        lo_v=pltpu.VMEM((NSUB * BINS,), jnp.float32),
        buf_v=pltpu.VMEM((WPB * BINS,), jnp.float32),
        below_v=pltpu.VMEM((NX,), jnp.float32),
        st_v=pltpu.VMEM((32,), jnp.float32),
        sml_v=pltpu.VMEM((32,), jnp.float32),
        out_v=pltpu.VMEM((16,), jnp.float32),
        sem=pltpu.SemaphoreType.DMA,
        val_sh=pltpu.VMEM_SHARED((NSUB * 32,), jnp.float32),
        rnk_sh=pltpu.VMEM_SHARED((NSUB * 32,), jnp.float32),
        hi_sh=pltpu.VMEM_SHARED((NSUB * BINS,), jnp.float32),
        lo_sh=pltpu.VMEM_SHARED((NSUB * BINS,), jnp.float32),
        st_sh=pltpu.VMEM_SHARED((NSUB * 32,), jnp.float32),
    ),
)
def _chamfer_sc(cu_hbm, tp_hbm, out_hbm, *, cu_all, cu_v, cs_v, cpad_v, rlo_v,
                y_v, hi_v, lo_v, buf_v, below_v, st_v, sml_v, out_v,
                sem, val_sh, rnk_sh, hi_sh, lo_sh, st_sh):
    _chamfer_body(cu_hbm, tp_hbm, out_hbm, cu_all, cu_v, cs_v, cpad_v, rlo_v,
                  y_v, hi_v, lo_v, buf_v, below_v, st_v, sml_v, out_v,
                  sem, val_sh, rnk_sh, hi_sh, lo_sh, st_sh)


def kernel(bin_centers, target):
    if bin_centers.ndim == 1:
        bin_centers = bin_centers[None, :]
    cu = bin_centers.astype(jnp.float32)
    tp = target.astype(jnp.float32)
    out = _chamfer_sc(cu, tp)
    return jnp.sum(out)
